# baseline (speedup 1.0000x reference)
import functools

import jax
import jax.numpy as jnp
from jax import lax
from jax.experimental import pallas as pl
from jax.experimental.pallas import tpu as pltpu
from jax.experimental.pallas import tpu_sc as plsc

_NC = 2
_NS = 16
_NW = _NC * _NS

_L = 16
_ROW = 128
_CR = 8
_CHUNK = _CR * _ROW

_RSQRT_MAGIC = 0x5F3759DF
_RNE_MAGIC = 12582912.0


def _sc_body(nchunks, trips, s_hbm, r_hbm, tx_hbm, ty_hbm, tz_hbm, box_hbm,
             out_x, out_y, out_z, out_r,
             sidx0, ridx0, ax0, ay0, az0, qx0, qy0, qz0,
             sidx1, ridx1, ax1, ay1, az1, qx1, qy1, qz1,
             px_v, py_v, pz_v, rr_v, box_v, sem0, sem1):
    wid = lax.axis_index("s") * _NC + lax.axis_index("c")
    sidx = (sidx0, sidx1)
    ridx = (ridx0, ridx1)
    abuf = ((ax0, ay0, az0), (ax1, ay1, az1))
    qbuf = ((qx0, qy0, qz0), (qx1, qy1, qz1))
    sem = (sem0, sem1)
    tabs = (tx_hbm, ty_hbm, tz_hbm)

    pltpu.sync_copy(box_hbm, box_v)
    bx = box_v[0, :]
    by = box_v[1, :]
    bz = box_v[2, :]
    ibx = box_v[3, :]
    iby = box_v[4, :]
    ibz = box_v[5, :]

    mrne = jnp.full((_L,), _RNE_MAGIC, jnp.float32)
    half = jnp.full((_L,), 0.5, jnp.float32)
    c15 = jnp.full((_L,), 1.5, jnp.float32)
    magic = jnp.full((_L,), _RSQRT_MAGIC, jnp.int32)

    def prefetch(c, g):
        row0 = _CR * c
        pltpu.sync_copy(s_hbm.at[pl.ds(row0, _CR)], sidx[g])
        pltpu.sync_copy(r_hbm.at[pl.ds(row0, _CR)], ridx[g])

        def fire(j, carry):
            sl = pl.ds(j * _ROW, _ROW)
            for k in range(3):
                pltpu.async_copy(tabs[k].at[sidx[g].at[j]],
                                 abuf[g][k].at[sl], sem[g])
                pltpu.async_copy(tabs[k].at[ridx[g].at[j]],
                                 qbuf[g][k].at[sl], sem[g])
            return carry

        lax.fori_loop(0, _CR, fire, 0)

    def drain(g):
        for k in range(3):
            pltpu.make_async_copy(tx_hbm.at[pl.ds(0, _CHUNK)],
                                  abuf[g][k], sem[g]).wait()
            pltpu.make_async_copy(tx_hbm.at[pl.ds(0, _CHUNK)],
                                  qbuf[g][k], sem[g]).wait()

    @pl.when(wid < nchunks)
    def _():
        prefetch(wid, 0)

    def process(c, g):
        @pl.when(c < nchunks)
        def _():
            drain(g)
            av, qv = abuf[g], qbuf[g]

            def step(i, carry2):
                sl = pl.ds(i * _L, _L)
                dx = qv[0][sl] - av[0][sl]
                dy = qv[1][sl] - av[1][sl]
                dz = qv[2][sl] - av[2][sl]
                nx = (dx * ibx + mrne) - mrne
                ny = (dy * iby + mrne) - mrne
                nz = (dz * ibz + mrne) - mrne
                dx = dx - nx * bx
                dy = dy - ny * by
                dz = dz - nz * bz
                s = dx * dx + dy * dy + dz * dz
                yi = magic - lax.shift_right_logical(plsc.bitcast(s, jnp.int32), 1)
                y = plsc.bitcast(yi, jnp.float32)
                hs = s * half
                y = y * (c15 - hs * y * y)
                y = y * (c15 - hs * y * y)
                y = y * (c15 - hs * y * y)
                rr = s * y
                px_v[sl] = dx
                py_v[sl] = dy
                pz_v[sl] = dz
                rr_v[sl] = rr
                return carry2

            lax.fori_loop(0, _CHUNK // _L, step, 0, unroll=2)

            base = c * _CHUNK
            pltpu.sync_copy(px_v, out_x.at[pl.ds(base, _CHUNK)])
            pltpu.sync_copy(py_v, out_y.at[pl.ds(base, _CHUNK)])
            pltpu.sync_copy(pz_v, out_z.at[pl.ds(base, _CHUNK)])
            pltpu.sync_copy(rr_v, out_r.at[pl.ds(base, _CHUNK)])

    def pair_body(u, carry):
        for g in (0, 1):
            t = 2 * u + g
            c = wid + _NW * t
            cn = c + _NW

            @pl.when(cn < nchunks)
            def _():
                prefetch(cn, 1 - g)

            process(c, g)
        return carry

    lax.fori_loop(0, (trips + 1) // 2, pair_body, 0)


def kernel(coords, boxsize, senders, receivers):
    n_edges = senders.shape[0]
    assert n_edges % _CHUNK == 0
    nchunks = n_edges // _CHUNK
    trips = (nchunks + _NW - 1) // _NW

    s32 = senders.astype(jnp.int32).reshape(n_edges // _ROW, _ROW)
    r32 = receivers.astype(jnp.int32).reshape(n_edges // _ROW, _ROW)
    cf = coords.astype(jnp.float32)
    tx = cf[:, 0] + 0.0
    ty = cf[:, 1] + 0.0
    tz = cf[:, 2] + 0.0
    box3 = boxsize.astype(jnp.float32).reshape(3)
    rows = [box3[0], box3[1], box3[2], 1.0 / box3[0], 1.0 / box3[1],
            1.0 / box3[2], jnp.float32(0.0), jnp.float32(0.0)]
    box_tab = jnp.stack([jnp.full((_L,), v, jnp.float32) for v in rows])

    mesh = plsc.VectorSubcoreMesh(core_axis_name="c", subcore_axis_name="s")
    buf = lambda: pltpu.VMEM((_CHUNK,), jnp.float32)
    idxbuf = lambda: pltpu.VMEM((_CR, _ROW), jnp.int32)
    f = functools.partial(
        pl.kernel,
        mesh=mesh,
        compiler_params=pltpu.CompilerParams(
            needs_layout_passes=False, use_tc_tiling_on_sc=False),
        out_type=[
            jax.ShapeDtypeStruct((n_edges,), jnp.float32),
            jax.ShapeDtypeStruct((n_edges,), jnp.float32),
            jax.ShapeDtypeStruct((n_edges,), jnp.float32),
            jax.ShapeDtypeStruct((n_edges,), jnp.float32),
        ],
        scratch_types=(
            [idxbuf(), idxbuf()] + [buf() for _ in range(6)]
            + [idxbuf(), idxbuf()] + [buf() for _ in range(6)]
            + [buf() for _ in range(4)]
            + [pltpu.VMEM((8, _L), jnp.float32),
               pltpu.SemaphoreType.DMA, pltpu.SemaphoreType.DMA]
        ),
    )(functools.partial(_sc_body, nchunks, trips))
    px, py, pz, rr = f(s32, r32, tx, ty, tz, box_tab)
    rx = jnp.stack([px, py, pz], axis=1)
    return (rr.reshape(n_edges, 1), rx)


# speedup vs baseline: 1.7986x; 1.7986x over previous
import functools

import jax
import jax.numpy as jnp
from jax import lax
from jax.experimental import pallas as pl
from jax.experimental.pallas import tpu as pltpu
from jax.experimental.pallas import tpu_sc as plsc

_NC = 2
_NS = 16
_NW = _NC * _NS

_L = 16
_ROW = 128
_CR = 8
_CHUNK = _CR * _ROW

_RSQRT_MAGIC = 0x5F3759DF
_RNE_MAGIC = 12582912.0


def _sc_body(nchunks, trips, s_hbm, r_hbm, tab_hbm, box_hbm,
             out_x, out_y, out_z, out_r,
             sidx0, ridx0, a0, b0, sidx1, ridx1, a1, b1,
             px_v, py_v, pz_v, rr_v, box_v, spm, sem0, sem1):
    wid = lax.axis_index("s") * _NC + lax.axis_index("c")
    sidx = (sidx0, sidx1)
    ridx = (ridx0, ridx1)
    a = (a0, a1)
    b = (b0, b1)
    sem = (sem0, sem1)

    @pl.when(lax.axis_index("s") == 0)
    def _():
        pltpu.sync_copy(tab_hbm, spm)

    plsc.subcore_barrier()

    pltpu.sync_copy(box_hbm, box_v)
    bx = box_v[0, :]
    by = box_v[1, :]
    bz = box_v[2, :]
    ibx = box_v[3, :]
    iby = box_v[4, :]
    ibz = box_v[5, :]

    iota = lax.iota(jnp.int32, _L)
    k0 = jnp.zeros((_L,), jnp.int32)
    k1 = jnp.full((_L,), 1, jnp.int32)
    k2 = jnp.full((_L,), 2, jnp.int32)
    mrne = jnp.full((_L,), _RNE_MAGIC, jnp.float32)
    half = jnp.full((_L,), 0.5, jnp.float32)
    c15 = jnp.full((_L,), 1.5, jnp.float32)
    magic = jnp.full((_L,), _RSQRT_MAGIC, jnp.int32)

    def prefetch(c, g):
        row0 = _CR * c
        pltpu.sync_copy(s_hbm.at[pl.ds(row0, _CR)], sidx[g])
        pltpu.sync_copy(r_hbm.at[pl.ds(row0, _CR)], ridx[g])
        for j in range(_CR):
            pltpu.async_copy(spm.at[sidx[g].at[j]],
                             a[g].at[pl.ds(j * _ROW, _ROW)], sem[g])
            pltpu.async_copy(spm.at[ridx[g].at[j]],
                             b[g].at[pl.ds(j * _ROW, _ROW)], sem[g])

    def drain(g):
        for j in range(_CR):
            pltpu.make_async_copy(spm.at[sidx[g].at[j]],
                                  a[g].at[pl.ds(j * _ROW, _ROW)], sem[g]).wait()
            pltpu.make_async_copy(spm.at[ridx[g].at[j]],
                                  b[g].at[pl.ds(j * _ROW, _ROW)], sem[g]).wait()

    @pl.when(wid < nchunks)
    def _():
        prefetch(wid, 0)

    def process(c, g, a_v, b_v):
        @pl.when(c < nchunks)
        def _():
            drain(g)

            def step(i, carry2):
                e16 = i * _L + iota
                ax = plsc.load_gather(a_v, [e16, k0])
                ay = plsc.load_gather(a_v, [e16, k1])
                az = plsc.load_gather(a_v, [e16, k2])
                qx = plsc.load_gather(b_v, [e16, k0])
                qy = plsc.load_gather(b_v, [e16, k1])
                qz = plsc.load_gather(b_v, [e16, k2])
                dx = qx - ax
                dy = qy - ay
                dz = qz - az
                nx = (dx * ibx + mrne) - mrne
                ny = (dy * iby + mrne) - mrne
                nz = (dz * ibz + mrne) - mrne
                dx = dx - nx * bx
                dy = dy - ny * by
                dz = dz - nz * bz
                s = dx * dx + dy * dy + dz * dz
                yi = magic - lax.shift_right_logical(plsc.bitcast(s, jnp.int32), 1)
                y = plsc.bitcast(yi, jnp.float32)
                hs = s * half
                y = y * (c15 - hs * y * y)
                y = y * (c15 - hs * y * y)
                y = y * (c15 - hs * y * y)
                rr = s * y
                sl = pl.ds(i * _L, _L)
                px_v[sl] = dx
                py_v[sl] = dy
                pz_v[sl] = dz
                rr_v[sl] = rr
                return carry2

            lax.fori_loop(0, _CHUNK // _L, step, 0, unroll=2)

            base = c * _CHUNK
            pltpu.sync_copy(px_v, out_x.at[pl.ds(base, _CHUNK)])
            pltpu.sync_copy(py_v, out_y.at[pl.ds(base, _CHUNK)])
            pltpu.sync_copy(pz_v, out_z.at[pl.ds(base, _CHUNK)])
            pltpu.sync_copy(rr_v, out_r.at[pl.ds(base, _CHUNK)])

    def pair_body(u, carry):
        for g in (0, 1):
            t = 2 * u + g
            c = wid + _NW * t
            cn = c + _NW

            @pl.when(cn < nchunks)
            def _():
                prefetch(cn, 1 - g)

            process(c, g, a[g], b[g])
        return carry

    lax.fori_loop(0, (trips + 1) // 2, pair_body, 0)


def kernel(coords, boxsize, senders, receivers):
    n_edges = senders.shape[0]
    assert n_edges % _CHUNK == 0
    nchunks = n_edges // _CHUNK
    trips = (nchunks + _NW - 1) // _NW

    s32 = senders.astype(jnp.int32).reshape(n_edges // _ROW, _ROW)
    r32 = receivers.astype(jnp.int32).reshape(n_edges // _ROW, _ROW)
    tab = jnp.pad(coords.astype(jnp.float32), ((0, 0), (0, 5)))
    box3 = boxsize.astype(jnp.float32).reshape(3)
    rows = [box3[0], box3[1], box3[2], 1.0 / box3[0], 1.0 / box3[1],
            1.0 / box3[2], jnp.float32(0.0), jnp.float32(0.0)]
    box_tab = jnp.stack([jnp.full((_L,), v, jnp.float32) for v in rows])

    mesh = plsc.VectorSubcoreMesh(core_axis_name="c", subcore_axis_name="s")
    f = functools.partial(
        pl.kernel,
        mesh=mesh,
        compiler_params=pltpu.CompilerParams(
            needs_layout_passes=False, use_tc_tiling_on_sc=False),
        out_type=[
            jax.ShapeDtypeStruct((n_edges,), jnp.float32),
            jax.ShapeDtypeStruct((n_edges,), jnp.float32),
            jax.ShapeDtypeStruct((n_edges,), jnp.float32),
            jax.ShapeDtypeStruct((n_edges,), jnp.float32),
        ],
        scratch_types=[
            pltpu.VMEM((_CR, _ROW), jnp.int32),
            pltpu.VMEM((_CR, _ROW), jnp.int32),
            pltpu.VMEM((_CHUNK, 8), jnp.float32),
            pltpu.VMEM((_CHUNK, 8), jnp.float32),
            pltpu.VMEM((_CR, _ROW), jnp.int32),
            pltpu.VMEM((_CR, _ROW), jnp.int32),
            pltpu.VMEM((_CHUNK, 8), jnp.float32),
            pltpu.VMEM((_CHUNK, 8), jnp.float32),
            pltpu.VMEM((_CHUNK,), jnp.float32),
            pltpu.VMEM((_CHUNK,), jnp.float32),
            pltpu.VMEM((_CHUNK,), jnp.float32),
            pltpu.VMEM((_CHUNK,), jnp.float32),
            pltpu.VMEM((8, _L), jnp.float32),
            pltpu.VMEM_SHARED((coords.shape[0], 8), jnp.float32),
            pltpu.SemaphoreType.DMA,
            pltpu.SemaphoreType.DMA,
        ],
    )(functools.partial(_sc_body, nchunks, trips))
    px, py, pz, rr = f(s32, r32, tab, box_tab)
    rx = jnp.stack([px, py, pz], axis=1)
    return (rr.reshape(n_edges, 1), rx)


# speedup vs baseline: 2.0658x; 1.1486x over previous
import functools

import jax
import jax.numpy as jnp
from jax import lax
from jax.experimental import pallas as pl
from jax.experimental.pallas import tpu as pltpu
from jax.experimental.pallas import tpu_sc as plsc

_NC = 2
_NS = 16
_NW = _NC * _NS

_L = 16
_ROW = 128
_CR = 8
_CHUNK = _CR * _ROW

_RSQRT_MAGIC = 0x5F3759DF
_RNE_MAGIC = 12582912.0


def _sc_body(nchunks, trips, s_hbm, r_hbm, tab_hbm, box_hbm,
             out_x, out_y, out_z, out_r,
             sidx0, ridx0, a0, b0, sidx1, ridx1, a1, b1,
             px0, py0, pz0, rr0, px1, py1, pz1, rr1,
             box_v, spm, sem0, sem1, semo0, semo1):
    wid = lax.axis_index("s") * _NC + lax.axis_index("c")
    sidx = (sidx0, sidx1)
    ridx = (ridx0, ridx1)
    a = (a0, a1)
    b = (b0, b1)
    outs = ((px0, py0, pz0, rr0), (px1, py1, pz1, rr1))
    sem = (sem0, sem1)
    semo = (semo0, semo1)

    @pl.when(lax.axis_index("s") == 0)
    def _():
        pltpu.sync_copy(tab_hbm, spm)

    plsc.subcore_barrier()

    pltpu.sync_copy(box_hbm, box_v)
    bx = box_v[0, :]
    by = box_v[1, :]
    bz = box_v[2, :]
    ibx = box_v[3, :]
    iby = box_v[4, :]
    ibz = box_v[5, :]

    iota = lax.iota(jnp.int32, _L)
    k0 = jnp.zeros((_L,), jnp.int32)
    k1 = jnp.full((_L,), 1, jnp.int32)
    k2 = jnp.full((_L,), 2, jnp.int32)
    mrne = jnp.full((_L,), _RNE_MAGIC, jnp.float32)
    half = jnp.full((_L,), 0.5, jnp.float32)
    c15 = jnp.full((_L,), 1.5, jnp.float32)
    magic = jnp.full((_L,), _RSQRT_MAGIC, jnp.int32)

    def prefetch(c, g):
        row0 = _CR * c
        pltpu.sync_copy(s_hbm.at[pl.ds(row0, _CR)], sidx[g])
        pltpu.sync_copy(r_hbm.at[pl.ds(row0, _CR)], ridx[g])
        for j in range(_CR):
            pltpu.async_copy(spm.at[sidx[g].at[j]],
                             a[g].at[pl.ds(j * _ROW, _ROW)], sem[g])
            pltpu.async_copy(spm.at[ridx[g].at[j]],
                             b[g].at[pl.ds(j * _ROW, _ROW)], sem[g])

    def drain(g):
        for j in range(_CR):
            pltpu.make_async_copy(spm.at[sidx[g].at[j]],
                                  a[g].at[pl.ds(j * _ROW, _ROW)], sem[g]).wait()
            pltpu.make_async_copy(spm.at[ridx[g].at[j]],
                                  b[g].at[pl.ds(j * _ROW, _ROW)], sem[g]).wait()

    @pl.when(wid < nchunks)
    def _():
        prefetch(wid, 0)

    def drain_outs(g, base):
        hbms = (out_x, out_y, out_z, out_r)
        for k in range(4):
            pltpu.make_async_copy(outs[g][k],
                                  hbms[k].at[pl.ds(base, _CHUNK)],
                                  semo[g]).wait()

    def process(t, c, g, a_v, b_v):
        @pl.when(c < nchunks)
        def _():
            @pl.when(t >= 2)
            def _():
                drain_outs(g, 0)

            drain(g)
            px_v, py_v, pz_v, rr_v = outs[g]

            def step(i, carry2):
                e16 = i * _L + iota
                ax = plsc.load_gather(a_v, [e16, k0])
                ay = plsc.load_gather(a_v, [e16, k1])
                az = plsc.load_gather(a_v, [e16, k2])
                qx = plsc.load_gather(b_v, [e16, k0])
                qy = plsc.load_gather(b_v, [e16, k1])
                qz = plsc.load_gather(b_v, [e16, k2])
                dx = qx - ax
                dy = qy - ay
                dz = qz - az
                nx = (dx * ibx + mrne) - mrne
                ny = (dy * iby + mrne) - mrne
                nz = (dz * ibz + mrne) - mrne
                dx = dx - nx * bx
                dy = dy - ny * by
                dz = dz - nz * bz
                s = dx * dx + dy * dy + dz * dz
                yi = magic - lax.shift_right_logical(plsc.bitcast(s, jnp.int32), 1)
                y = plsc.bitcast(yi, jnp.float32)
                hs = s * half
                y = y * (c15 - hs * y * y)
                y = y * (c15 - hs * y * y)
                rr = s * y
                sl = pl.ds(i * _L, _L)
                px_v[sl] = dx
                py_v[sl] = dy
                pz_v[sl] = dz
                rr_v[sl] = rr
                return carry2

            lax.fori_loop(0, _CHUNK // _L, step, 0, unroll=2)

            base = c * _CHUNK
            pltpu.async_copy(px_v, out_x.at[pl.ds(base, _CHUNK)], semo[g])
            pltpu.async_copy(py_v, out_y.at[pl.ds(base, _CHUNK)], semo[g])
            pltpu.async_copy(pz_v, out_z.at[pl.ds(base, _CHUNK)], semo[g])
            pltpu.async_copy(rr_v, out_r.at[pl.ds(base, _CHUNK)], semo[g])

    def pair_body(u, carry):
        for g in (0, 1):
            t = 2 * u + g
            c = wid + _NW * t
            cn = c + _NW

            @pl.when(cn < nchunks)
            def _():
                prefetch(cn, 1 - g)

            process(t, c, g, a[g], b[g])
        return carry

    lax.fori_loop(0, (trips + 1) // 2, pair_body, 0)

    np_w = (nchunks - wid + _NW - 1) // _NW

    for back in (1, 2):
        @pl.when(np_w >= back)
        def _(back=back):
            par = lax.rem(np_w - back, 2)

            @pl.when(par == 0)
            def _():
                drain_outs(0, 0)

            @pl.when(par == 1)
            def _():
                drain_outs(1, 0)


def kernel(coords, boxsize, senders, receivers):
    n_edges = senders.shape[0]
    assert n_edges % _CHUNK == 0
    nchunks = n_edges // _CHUNK
    trips = (nchunks + _NW - 1) // _NW

    s32 = senders.astype(jnp.int32).reshape(n_edges // _ROW, _ROW)
    r32 = receivers.astype(jnp.int32).reshape(n_edges // _ROW, _ROW)
    tab = jnp.pad(coords.astype(jnp.float32), ((0, 0), (0, 5)))
    box3 = boxsize.astype(jnp.float32).reshape(3)
    rows = [box3[0], box3[1], box3[2], 1.0 / box3[0], 1.0 / box3[1],
            1.0 / box3[2], jnp.float32(0.0), jnp.float32(0.0)]
    box_tab = jnp.stack([jnp.full((_L,), v, jnp.float32) for v in rows])

    mesh = plsc.VectorSubcoreMesh(core_axis_name="c", subcore_axis_name="s")
    f = functools.partial(
        pl.kernel,
        mesh=mesh,
        compiler_params=pltpu.CompilerParams(
            needs_layout_passes=False, use_tc_tiling_on_sc=False),
        out_type=[
            jax.ShapeDtypeStruct((n_edges,), jnp.float32),
            jax.ShapeDtypeStruct((n_edges,), jnp.float32),
            jax.ShapeDtypeStruct((n_edges,), jnp.float32),
            jax.ShapeDtypeStruct((n_edges,), jnp.float32),
        ],
        scratch_types=[
            pltpu.VMEM((_CR, _ROW), jnp.int32),
            pltpu.VMEM((_CR, _ROW), jnp.int32),
            pltpu.VMEM((_CHUNK, 8), jnp.float32),
            pltpu.VMEM((_CHUNK, 8), jnp.float32),
            pltpu.VMEM((_CR, _ROW), jnp.int32),
            pltpu.VMEM((_CR, _ROW), jnp.int32),
            pltpu.VMEM((_CHUNK, 8), jnp.float32),
            pltpu.VMEM((_CHUNK, 8), jnp.float32),
            pltpu.VMEM((_CHUNK,), jnp.float32),
            pltpu.VMEM((_CHUNK,), jnp.float32),
            pltpu.VMEM((_CHUNK,), jnp.float32),
            pltpu.VMEM((_CHUNK,), jnp.float32),
            pltpu.VMEM((_CHUNK,), jnp.float32),
            pltpu.VMEM((_CHUNK,), jnp.float32),
            pltpu.VMEM((_CHUNK,), jnp.float32),
            pltpu.VMEM((_CHUNK,), jnp.float32),
            pltpu.VMEM((8, _L), jnp.float32),
            pltpu.VMEM_SHARED((coords.shape[0], 8), jnp.float32),
            pltpu.SemaphoreType.DMA,
            pltpu.SemaphoreType.DMA,
            pltpu.SemaphoreType.DMA,
            pltpu.SemaphoreType.DMA,
        ],
    )(functools.partial(_sc_body, nchunks, trips))
    px, py, pz, rr = f(s32, r32, tab, box_tab)
    rx = jnp.stack([px, py, pz], axis=1)
    return (rr.reshape(n_edges, 1), rx)


# speedup vs baseline: 2.4173x; 1.1701x over previous
import functools

import jax
import jax.numpy as jnp
from jax import lax
from jax.experimental import pallas as pl
from jax.experimental.pallas import tpu as pltpu
from jax.experimental.pallas import tpu_sc as plsc

_NC = 2
_NS = 16
_NW = _NC * _NS

_L = 16
_ROW = 128
_CR = 10
_CHUNK = _CR * _ROW

_RSQRT_MAGIC = 0x5F3759DF
_RNE_MAGIC = 12582912.0


def _sc_body(nchunks, trips, s_hbm, r_hbm, tab_hbm, box_hbm,
             out_x, out_y, out_z, out_r,
             sidx0, ridx0, a0, b0, sidx1, ridx1, a1, b1,
             px0, py0, pz0, rr0, px1, py1, pz1, rr1,
             box_v, spm, sem0, sem1, semo0, semo1, semi0, semi1):
    wid = lax.axis_index("s") * _NC + lax.axis_index("c")
    sidx = (sidx0, sidx1)
    ridx = (ridx0, ridx1)
    a = (a0, a1)
    b = (b0, b1)
    outs = ((px0, py0, pz0, rr0), (px1, py1, pz1, rr1))
    sem = (sem0, sem1)
    semo = (semo0, semo1)
    semi = (semi0, semi1)

    @pl.when(lax.axis_index("s") == 0)
    def _():
        pltpu.sync_copy(tab_hbm, spm)

    plsc.subcore_barrier()

    pltpu.sync_copy(box_hbm, box_v)
    bx = box_v[0, :]
    by = box_v[1, :]
    bz = box_v[2, :]
    ibx = box_v[3, :]
    iby = box_v[4, :]
    ibz = box_v[5, :]

    iota = lax.iota(jnp.int32, _L)
    k0 = jnp.zeros((_L,), jnp.int32)
    k1 = jnp.full((_L,), 1, jnp.int32)
    k2 = jnp.full((_L,), 2, jnp.int32)
    mrne = jnp.full((_L,), _RNE_MAGIC, jnp.float32)
    half = jnp.full((_L,), 0.5, jnp.float32)
    c15 = jnp.full((_L,), 1.5, jnp.float32)
    magic = jnp.full((_L,), _RSQRT_MAGIC, jnp.int32)

    def idx_load(c, h):
        row0 = _CR * c
        pltpu.async_copy(s_hbm.at[pl.ds(row0, _CR)], sidx[h], semi[h])
        pltpu.async_copy(r_hbm.at[pl.ds(row0, _CR)], ridx[h], semi[h])

    def fire_gathers(c, g):
        row0 = _CR * c
        pltpu.make_async_copy(s_hbm.at[pl.ds(row0, _CR)], sidx[g],
                              semi[g]).wait()
        pltpu.make_async_copy(r_hbm.at[pl.ds(row0, _CR)], ridx[g],
                              semi[g]).wait()
        for j in range(_CR):
            pltpu.async_copy(spm.at[sidx[g].at[j]],
                             a[g].at[pl.ds(j * _ROW, _ROW)], sem[g])
            pltpu.async_copy(spm.at[ridx[g].at[j]],
                             b[g].at[pl.ds(j * _ROW, _ROW)], sem[g])

    def drain(g):
        for j in range(_CR):
            pltpu.make_async_copy(spm.at[sidx[g].at[j]],
                                  a[g].at[pl.ds(j * _ROW, _ROW)], sem[g]).wait()
            pltpu.make_async_copy(spm.at[ridx[g].at[j]],
                                  b[g].at[pl.ds(j * _ROW, _ROW)], sem[g]).wait()

    @pl.when(wid < nchunks)
    def _():
        idx_load(wid, 0)
        fire_gathers(wid, 0)

    @pl.when(wid + _NW < nchunks)
    def _():
        idx_load(wid + _NW, 1)

    def drain_outs(g, base):
        hbms = (out_x, out_y, out_z, out_r)
        for k in range(4):
            pltpu.make_async_copy(outs[g][k],
                                  hbms[k].at[pl.ds(base, _CHUNK)],
                                  semo[g]).wait()

    def process(t, c, g, a_v, b_v):
        @pl.when(c < nchunks)
        def _():
            @pl.when(t >= 2)
            def _():
                drain_outs(g, 0)

            drain(g)
            px_v, py_v, pz_v, rr_v = outs[g]

            def step(i, carry2):
                e16 = i * _L + iota
                ax = plsc.load_gather(a_v, [e16, k0])
                ay = plsc.load_gather(a_v, [e16, k1])
                az = plsc.load_gather(a_v, [e16, k2])
                qx = plsc.load_gather(b_v, [e16, k0])
                qy = plsc.load_gather(b_v, [e16, k1])
                qz = plsc.load_gather(b_v, [e16, k2])
                dx = qx - ax
                dy = qy - ay
                dz = qz - az
                nx = (dx * ibx + mrne) - mrne
                ny = (dy * iby + mrne) - mrne
                nz = (dz * ibz + mrne) - mrne
                dx = dx - nx * bx
                dy = dy - ny * by
                dz = dz - nz * bz
                s = dx * dx + dy * dy + dz * dz
                yi = magic - lax.shift_right_logical(plsc.bitcast(s, jnp.int32), 1)
                y = plsc.bitcast(yi, jnp.float32)
                hs = s * half
                y = y * (c15 - hs * y * y)
                y = y * (c15 - hs * y * y)
                rr = s * y
                sl = pl.ds(i * _L, _L)
                px_v[sl] = dx
                py_v[sl] = dy
                pz_v[sl] = dz
                rr_v[sl] = rr
                return carry2

            lax.fori_loop(0, _CHUNK // _L, step, 0, unroll=2)

            base = c * _CHUNK
            pltpu.async_copy(px_v, out_x.at[pl.ds(base, _CHUNK)], semo[g])
            pltpu.async_copy(py_v, out_y.at[pl.ds(base, _CHUNK)], semo[g])
            pltpu.async_copy(pz_v, out_z.at[pl.ds(base, _CHUNK)], semo[g])
            pltpu.async_copy(rr_v, out_r.at[pl.ds(base, _CHUNK)], semo[g])

    def pair_body(u, carry):
        for g in (0, 1):
            t = 2 * u + g
            c = wid + _NW * t
            cn = c + _NW
            cnn = cn + _NW

            @pl.when(cn < nchunks)
            def _():
                fire_gathers(cn, 1 - g)

            process(t, c, g, a[g], b[g])

            @pl.when(cnn < nchunks)
            def _():
                idx_load(cnn, g)
        return carry

    lax.fori_loop(0, (trips + 1) // 2, pair_body, 0)

    np_w = (nchunks - wid + _NW - 1) // _NW

    for back in (1, 2):
        @pl.when(np_w >= back)
        def _(back=back):
            par = lax.rem(np_w - back, 2)

            @pl.when(par == 0)
            def _():
                drain_outs(0, 0)

            @pl.when(par == 1)
            def _():
                drain_outs(1, 0)


def kernel(coords, boxsize, senders, receivers):
    n_edges = senders.shape[0]
    assert n_edges % _CHUNK == 0
    nchunks = n_edges // _CHUNK
    trips = (nchunks + _NW - 1) // _NW

    s32 = senders.astype(jnp.int32).reshape(n_edges // _ROW, _ROW)
    r32 = receivers.astype(jnp.int32).reshape(n_edges // _ROW, _ROW)
    tab = jnp.pad(coords.astype(jnp.float32), ((0, 0), (0, 5)))
    box3 = boxsize.astype(jnp.float32).reshape(3)
    rows = [box3[0], box3[1], box3[2], 1.0 / box3[0], 1.0 / box3[1],
            1.0 / box3[2], jnp.float32(0.0), jnp.float32(0.0)]
    box_tab = jnp.stack([jnp.full((_L,), v, jnp.float32) for v in rows])

    mesh = plsc.VectorSubcoreMesh(core_axis_name="c", subcore_axis_name="s")
    f = functools.partial(
        pl.kernel,
        mesh=mesh,
        compiler_params=pltpu.CompilerParams(
            needs_layout_passes=False, use_tc_tiling_on_sc=False),
        out_type=[
            jax.ShapeDtypeStruct((n_edges,), jnp.float32),
            jax.ShapeDtypeStruct((n_edges,), jnp.float32),
            jax.ShapeDtypeStruct((n_edges,), jnp.float32),
            jax.ShapeDtypeStruct((n_edges,), jnp.float32),
        ],
        scratch_types=[
            pltpu.VMEM((_CR, _ROW), jnp.int32),
            pltpu.VMEM((_CR, _ROW), jnp.int32),
            pltpu.VMEM((_CHUNK, 8), jnp.float32),
            pltpu.VMEM((_CHUNK, 8), jnp.float32),
            pltpu.VMEM((_CR, _ROW), jnp.int32),
            pltpu.VMEM((_CR, _ROW), jnp.int32),
            pltpu.VMEM((_CHUNK, 8), jnp.float32),
            pltpu.VMEM((_CHUNK, 8), jnp.float32),
            pltpu.VMEM((_CHUNK,), jnp.float32),
            pltpu.VMEM((_CHUNK,), jnp.float32),
            pltpu.VMEM((_CHUNK,), jnp.float32),
            pltpu.VMEM((_CHUNK,), jnp.float32),
            pltpu.VMEM((_CHUNK,), jnp.float32),
            pltpu.VMEM((_CHUNK,), jnp.float32),
            pltpu.VMEM((_CHUNK,), jnp.float32),
            pltpu.VMEM((_CHUNK,), jnp.float32),
            pltpu.VMEM((8, _L), jnp.float32),
            pltpu.VMEM_SHARED((coords.shape[0], 8), jnp.float32),
            pltpu.SemaphoreType.DMA,
            pltpu.SemaphoreType.DMA,
            pltpu.SemaphoreType.DMA,
            pltpu.SemaphoreType.DMA,
            pltpu.SemaphoreType.DMA,
            pltpu.SemaphoreType.DMA,
        ],
    )(functools.partial(_sc_body, nchunks, trips))
    px, py, pz, rr = f(s32, r32, tab, box_tab)
    rx = jnp.stack([px, py, pz], axis=1)
    return (rr.reshape(n_edges, 1), rx)
